# Initial kernel scaffold; baseline (speedup 1.0000x reference)
import jax
import jax.numpy as jnp
from jax.experimental import pallas as pl

OUT_D = 4096
ROWS = 5
BB = 8


def _body(floats_ref, out_ref):
    begin = floats_ref[:, 0:1]
    end = floats_ref[:, 1:2]
    i = jax.lax.broadcasted_iota(jnp.float32, (BB, OUT_D), 1)
    lin = begin + (i * (end - begin)) * (1.0 / 4095.0)
    zero = jnp.zeros_like(lin)
    out_ref[:, 0, :] = jnp.maximum(0.0, jnp.minimum(lin, 2.0 - lin))
    out_ref[:, 1, :] = jnp.maximum(0.0, lin - 1.0)
    out_ref[:, 2, :] = zero
    out_ref[:, 3, :] = zero
    out_ref[:, 4, :] = jnp.maximum(0.0, -lin)


def kernel(floats):
    b = floats.shape[0]
    mats = pl.pallas_call(
        _body,
        grid=(b // BB,),
        in_specs=[pl.BlockSpec((BB, 2), lambda i: (i, 0))],
        out_specs=pl.BlockSpec((BB, ROWS, OUT_D), lambda i: (i, 0, 0)),
        out_shape=jax.ShapeDtypeStruct((b, ROWS, OUT_D), jnp.float32),
    )(floats)
    return (mats, jnp.ones((b, 1), jnp.float32))


# speedup vs baseline: 223.3793x; 223.3793x over previous
import jax
import jax.numpy as jnp
from jax.experimental import pallas as pl

OUT_D = 4096
ROWS = 5
BB = 8


def _body(floats_ref, out_ref):
    begin = floats_ref[:, 0:1]
    end = floats_ref[:, 1:2]
    i = jax.lax.broadcasted_iota(jnp.int32, (BB, OUT_D), 1).astype(jnp.float32)
    lin = begin + (i * (end - begin)) * (1.0 / 4095.0)
    zero = jnp.zeros_like(lin)
    out_ref[:, 0, :] = jnp.maximum(0.0, jnp.minimum(lin, 2.0 - lin))
    out_ref[:, 1, :] = jnp.maximum(0.0, lin - 1.0)
    out_ref[:, 2, :] = zero
    out_ref[:, 3, :] = zero
    out_ref[:, 4, :] = jnp.maximum(0.0, -lin)


def kernel(floats):
    b = floats.shape[0]
    mats = pl.pallas_call(
        _body,
        grid=(b // BB,),
        in_specs=[pl.BlockSpec((BB, 2), lambda i: (i, 0))],
        out_specs=pl.BlockSpec((BB, ROWS, OUT_D), lambda i: (i, 0, 0)),
        out_shape=jax.ShapeDtypeStruct((b, ROWS, OUT_D), jnp.float32),
    )(floats)
    return (mats, jnp.ones((b, 1), jnp.float32))
